# baseline (speedup 1.0000x reference)
import functools

import jax
import jax.numpy as jnp
from jax import lax
from jax.experimental import pallas as pl
from jax.experimental.pallas import tpu as pltpu
from jax.experimental.pallas import tpu_sc as plsc

ROWS_BLK = 16
LANES = 16


def kernel(x, pos_table):
    B, S, D = x.shape
    xf = x.reshape(B * S, D)
    n_blocks = (B * S) // ROWS_BLK
    pos_blocks = S // ROWS_BLK
    mesh = plsc.VectorSubcoreMesh(core_axis_name="c", subcore_axis_name="s")

    @functools.partial(
        pl.kernel,
        out_type=jax.ShapeDtypeStruct((B * S, D), x.dtype),
        mesh=mesh,
    )
    def sc_add(x_hbm, pos_hbm, o_hbm):
        def body(x_vmem, pos_vmem, o_vmem):
            @pl.loop(0, ROWS_BLK)
            def _(r):
                @pl.loop(0, D, step=LANES)
                def _(c):
                    o_vmem.at[r, pl.ds(c, LANES)][...] = (
                        x_vmem.at[r, pl.ds(c, LANES)][...]
                        + pos_vmem.at[r, pl.ds(c, LANES)][...]
                    )

        pltpu.emit_pipeline(
            body,
            grid=(n_blocks,),
            in_specs=[
                pl.BlockSpec((ROWS_BLK, D), index_map=lambda i: (i, 0)),
                pl.BlockSpec(
                    (ROWS_BLK, D),
                    index_map=lambda i: (lax.rem(i, pos_blocks), 0),
                ),
            ],
            out_specs=[pl.BlockSpec((ROWS_BLK, D), index_map=lambda i: (i, 0))],
            core_axis_name=("c", "s"),
            dimension_semantics=(pltpu.PARALLEL,),
        )(x_hbm, pos_hbm, o_hbm)

    return sc_add(xf, pos_table).reshape(B, S, D)


# speedup vs baseline: 1.1080x; 1.1080x over previous
import functools

import jax
import jax.numpy as jnp
from jax import lax
from jax.experimental import pallas as pl
from jax.experimental.pallas import tpu as pltpu
from jax.experimental.pallas import tpu_sc as plsc

ROWS_BLK = 16
LANES = 16


def kernel(x, pos_table):
    B, S, D = x.shape
    xf = x.reshape(B * S, D)
    n_blocks = (B * S) // ROWS_BLK
    pos_blocks = S // ROWS_BLK
    mesh = plsc.VectorSubcoreMesh(core_axis_name="c", subcore_axis_name="s")

    @functools.partial(
        pl.kernel,
        out_type=jax.ShapeDtypeStruct((B * S, D), x.dtype),
        mesh=mesh,
    )
    def sc_add(x_hbm, pos_hbm, o_hbm):
        def body(x_vmem, pos_vmem, o_vmem):
            @pl.loop(0, ROWS_BLK)
            def _(r):
                for c in range(0, D, LANES):
                    o_vmem.at[r, pl.ds(c, LANES)][...] = (
                        x_vmem.at[r, pl.ds(c, LANES)][...]
                        + pos_vmem.at[r, pl.ds(c, LANES)][...]
                    )

        pltpu.emit_pipeline(
            body,
            grid=(n_blocks,),
            in_specs=[
                pl.BlockSpec((ROWS_BLK, D), index_map=lambda i: (i, 0)),
                pl.BlockSpec(
                    (ROWS_BLK, D),
                    index_map=lambda i: (lax.rem(i, pos_blocks), 0),
                ),
            ],
            out_specs=[pl.BlockSpec((ROWS_BLK, D), index_map=lambda i: (i, 0))],
            core_axis_name=("c", "s"),
            dimension_semantics=(pltpu.PARALLEL,),
        )(x_hbm, pos_hbm, o_hbm)

    return sc_add(xf, pos_table).reshape(B, S, D)


# speedup vs baseline: 4.3513x; 3.9272x over previous
import jax
import jax.numpy as jnp
from jax.experimental import pallas as pl

S_BLK = 2048


def _add_kernel(x_ref, pos_ref, out_ref):
    out_ref[0] = x_ref[0] + pos_ref[...]


def kernel(x, pos_table):
    B, S, D = x.shape
    grid = (S // S_BLK, B)
    return pl.pallas_call(
        _add_kernel,
        grid=grid,
        in_specs=[
            pl.BlockSpec((1, S_BLK, D), lambda i, j: (j, i, 0)),
            pl.BlockSpec((S_BLK, D), lambda i, j: (i, 0)),
        ],
        out_specs=pl.BlockSpec((1, S_BLK, D), lambda i, j: (j, i, 0)),
        out_shape=jax.ShapeDtypeStruct((B, S, D), x.dtype),
    )(x, pos_table)
